# baseline (speedup 1.0000x reference)
import functools

import jax
import jax.numpy as jnp
from jax import lax
from jax.experimental import pallas as pl
from jax.experimental.pallas import tpu as pltpu
from jax.experimental.pallas import tpu_sc as plsc

N = 10000
E = 320000
NROW = 10240
TRASH = 10008
DC = 128
BATCH = 128
NB_ALL = 168
NB_HALF = 80
EPAD = 32 * NB_HALF * BATCH
NBTOT = EPAD // BATCH
EPAD2 = 2784 * BATCH
ROWS_PER_TILE = NROW // 16
DRAIN_CHUNK = 128

_mesh = plsc.VectorSubcoreMesh(core_axis_name="c", subcore_axis_name="s")


def _fill_rows(buf, nrows, ncols, value):
    vec = jnp.full((16,), value, jnp.float32)

    def body(i, _):
        for c in range(ncols // 16):
            buf[i, pl.ds(c * 16, 16)] = vec
        return 0

    lax.fori_loop(0, nrows, body, 0)


HALF = NROW // 2
ATRASH = 128
HROWS_PER_TILE = HALF // 16


def _chunk_pass(table, out, src_v, dst_v, bufs, acc, gsem,
                tile, cnt_b, row_base):
    for r0 in range(0, HROWS_PER_TILE, DRAIN_CHUNK):
        nrow = min(DRAIN_CHUNK, HROWS_PER_TILE - r0)
        row0 = tile * HROWS_PER_TILE + r0
        pltpu.sync_copy(table.at[pl.ds(row_base + row0, nrow)],
                        bufs[0].at[pl.ds(0, nrow)])
        pltpu.sync_copy(bufs[0].at[pl.ds(0, nrow)],
                        acc.at[pl.ds(row0, nrow)])
    plsc.subcore_barrier()

    rows_a, rows_b = bufs
    sem_a, sem_b = gsem

    def gather(j, buf, sem):
        return pltpu.make_async_copy(
            table.at[src_v.at[pl.ds(j * BATCH, BATCH)]], buf, sem)

    @pl.when(cnt_b > 0)
    def _():
        gather(0, rows_a, sem_a).start()

    def loop(k, _):
        ja = 2 * k
        jb = ja + 1
        gather(jb, rows_b, sem_b).start()
        gather(ja, rows_a, sem_a).wait()
        pltpu.sync_copy(rows_a, acc.at[dst_v.at[ja]], add=True)

        @pl.when(ja + 2 < cnt_b)
        def _():
            gather(ja + 2, rows_a, sem_a).start()

        gather(jb, rows_b, sem_b).wait()
        pltpu.sync_copy(rows_b, acc.at[dst_v.at[jb]], add=True)
        return 0

    lax.fori_loop(0, cnt_b // 2, loop, 0)

    @pl.when(cnt_b % 2 == 1)
    def _():
        jl = cnt_b - 1
        gather(jl, rows_a, sem_a).wait()
        pltpu.sync_copy(rows_a, acc.at[dst_v.at[jl]], add=True)

    plsc.subcore_barrier()

    for r0 in range(0, HROWS_PER_TILE, DRAIN_CHUNK):
        nrow = min(DRAIN_CHUNK, HROWS_PER_TILE - r0)
        row0 = tile * HROWS_PER_TILE + r0
        pltpu.sync_copy(acc.at[pl.ds(row0, nrow)],
                        bufs[0].at[pl.ds(0, nrow)])
        pltpu.sync_copy(bufs[0].at[pl.ds(0, nrow)],
                        out.at[pl.ds(row_base + row0, nrow)])


def _make_agg(num_chunks):
    @functools.partial(
        pl.kernel,
        out_type=[jax.ShapeDtypeStruct((NROW, DC), jnp.float32)]
        * num_chunks,
        mesh=_mesh,
        scratch_types=[
            pltpu.VMEM((16,), jnp.int32),
            pltpu.VMEM((NB_ALL * BATCH,), jnp.int32),
            pltpu.VMEM((NB_ALL, BATCH), jnp.int32),
            pltpu.VMEM((BATCH, DC), jnp.float32),
            pltpu.VMEM((BATCH, DC), jnp.float32),
            pltpu.VMEM_SHARED((HALF + ATRASH, DC), jnp.float32),
            pltpu.SemaphoreType.DMA,
            pltpu.SemaphoreType.DMA,
        ],
    )
    def agg(cnt_hbm, src_hbm, dst_hbm, *rest):
        tables = rest[:num_chunks]
        outs = rest[num_chunks:2 * num_chunks]
        (cnt_v, src_v, dst_v, rows_a, rows_b, acc, sem_a, sem_b) = \
            rest[2 * num_chunks:]
        core = lax.axis_index("c")
        tile = lax.axis_index("s")

        pltpu.sync_copy(cnt_hbm, cnt_v)
        n0 = cnt_v[...][0]
        lo8 = jnp.where(core == 0, 0, n0 // (8 * BATCH))
        hi = jnp.where(core == 0, (n0 + BATCH - 1) // BATCH, NBTOT)
        nbt8 = (hi - lo8 * 8 + 127) // 128
        lo_t = (lo8 + tile * nbt8) * 8
        cnt_t = jnp.maximum(0, jnp.minimum(nbt8 * 8, hi - lo_t))

        pltpu.sync_copy(src_hbm.at[pl.ds(lo_t * BATCH, NB_ALL * BATCH)],
                        src_v)
        pltpu.sync_copy(dst_hbm.at[pl.ds(lo_t, NB_ALL)], dst_v)

        row_base = core * HALF

        def remap(j, _):
            for c in range(BATCH // 16):
                d = dst_v[j, pl.ds(c * 16, 16)]
                local = d - row_base
                oob = (local < 0) | (local >= HALF)
                spread = HALF + tile * 8 + (local & 7)
                dst_v[j, pl.ds(c * 16, 16)] = jnp.where(oob, spread, local)
            return 0

        lax.fori_loop(0, NB_ALL, remap, 0)

        for cid in range(2):
            @pl.when(core == cid)
            def _(cid=cid):
                for ch in range(num_chunks):
                    _chunk_pass(tables[ch], outs[ch], src_v, dst_v,
                                (rows_a, rows_b), acc, (sem_a, sem_b),
                                tile, cnt_t, cid * HALF)

    return agg


def _make_deg():
    @functools.partial(
        pl.kernel,
        out_type=[jax.ShapeDtypeStruct((NROW, 16), jnp.float32)] * 2,
        mesh=_mesh,
        scratch_types=[
            pltpu.VMEM((NB_HALF, BATCH), jnp.int32),
            pltpu.VMEM((BATCH, 16), jnp.float32),
            pltpu.VMEM((BATCH, 16), jnp.float32),
            pltpu.VMEM_SHARED((NROW, 16), jnp.float32),
        ],
    )
    def deg(dst_hbm, out0, out1, dst_v, ones_v, stage_v, acc):
        core = lax.axis_index("c")
        tile = lax.axis_index("s")
        outs = (out0, out1)

        _fill_rows(ones_v, BATCH, 16, 1.0)

        base = (core * 16 + tile) * NB_HALF
        pltpu.sync_copy(dst_hbm.at[pl.ds(base, NB_HALF)], dst_v)

        vec = jnp.full((16,), 1.0, jnp.float32) * jnp.where(
            core == 0, 1.0, 0.0).astype(jnp.float32)

        def fill(i, _):
            stage_v[i, pl.ds(0, 16)] = vec
            return 0

        lax.fori_loop(0, BATCH, fill, 0)
        for r in range(ROWS_PER_TILE // DRAIN_CHUNK):
            row0 = tile * ROWS_PER_TILE + r * DRAIN_CHUNK
            pltpu.sync_copy(stage_v.at[pl.ds(0, DRAIN_CHUNK)],
                            acc.at[pl.ds(row0, DRAIN_CHUNK)])
        plsc.subcore_barrier()

        def loop(j, _):
            pltpu.sync_copy(ones_v, acc.at[dst_v.at[j]], add=True)
            return 0

        lax.fori_loop(0, NB_HALF, loop, 0)
        plsc.subcore_barrier()

        for cid in range(2):
            @pl.when(core == cid)
            def _(cid=cid):
                for r in range(ROWS_PER_TILE // DRAIN_CHUNK):
                    row0 = tile * ROWS_PER_TILE + r * DRAIN_CHUNK
                    pltpu.sync_copy(acc.at[pl.ds(row0, DRAIN_CHUNK)],
                                    stage_v.at[pl.ds(0, DRAIN_CHUNK)])
                    pltpu.sync_copy(stage_v.at[pl.ds(0, DRAIN_CHUNK)],
                                    outs[cid].at[pl.ds(row0, DRAIN_CHUNK)])

    return deg


_agg_1 = _make_agg(1)
_agg_4 = _make_agg(4)
_agg_2 = _make_agg(2)
_deg_kernel = _make_deg()


BM = 1024


def _row(d):
    return pl.BlockSpec((BM, d), lambda i: (i, 0))


def _full(shape):
    return pl.BlockSpec(shape, lambda i: tuple(0 for _ in shape))


def _tc_call(body, in_dims, full_shapes, out_dims):
    in_specs = ([_row(d) for d in in_dims]
                + [_row(16), _row(16)]
                + [_full(s) for s in full_shapes])
    return pl.pallas_call(
        body,
        grid=(NROW // BM,),
        in_specs=in_specs,
        out_specs=[_row(d) for d in out_dims],
        out_shape=[jax.ShapeDtypeStruct((NROW, d), jnp.float32)
                   for d in out_dims],
    )


def _scale(d0_ref, d1_ref):
    return lax.rsqrt(d0_ref[...] + d1_ref[...])[:, 0:1]


def _tca(x_ref, d0, d1, o0):
    o0[...] = x_ref[...] * _scale(d0, d1)


def _tcb(r0, d0, d1, w0, b0, w1, o0, o1, o2, o3):
    s = _scale(d0, d1)
    h = jnp.maximum(
        jnp.dot(r0[...] * s, w0[...], preferred_element_type=jnp.float32)
        + b0[...], 0.0)
    z = jnp.dot(h, w1[...], preferred_element_type=jnp.float32) * s
    for j, o in enumerate((o0, o1, o2, o3)):
        o[...] = z[:, j * 128:(j + 1) * 128]


def _tcc(r0, r1, r2, r3, d0, d1, b1, w2, o0, o1):
    s = _scale(d0, d1)
    z = None
    for j, r in enumerate((r0, r1, r2, r3)):
        h = jnp.maximum(r[...] * s + b1[..., j * 128:(j + 1) * 128], 0.0)
        p = jnp.dot(h, w2[j * 128:(j + 1) * 128, :],
                    preferred_element_type=jnp.float32)
        z = p if z is None else z + p
    z = z * s
    o0[...] = z[:, :128]
    o1[...] = z[:, 128:]


def _tcd(r0, r1, d0, d1, b2, w3p, o0):
    s = _scale(d0, d1)
    z = None
    for j, r in enumerate((r0, r1)):
        h = jnp.maximum(r[...] * s + b2[..., j * 128:(j + 1) * 128], 0.0)
        p = jnp.dot(h, w3p[j * 128:(j + 1) * 128, :],
                    preferred_element_type=jnp.float32)
        z = p if z is None else z + p
    o0[...] = z * s


def _tce(ra, d0, d1, b3p, o0):
    s = _scale(d0, d1)
    o0[...] = jnp.maximum(ra[...] * s + b3p[...], 0.0) * s


def _tcf(ra, d0, d1, w4, b4, o0):
    s = _scale(d0, d1)
    o0[...] = jnp.dot((ra[...] * s)[:, :64], w4[...],
                      preferred_element_type=jnp.float32) + b4[...]


def kernel(x, edge_index, W0, b0, W1, b1, W2, b2, W3, b3, W4, b4):
    src = edge_index[0]
    dst = edge_index[1]
    key = (dst >= HALF).astype(jnp.int32)
    n0 = (jnp.int32(E) - jnp.sum(key)).astype(jnp.int32)
    c1 = jnp.cumsum(key)
    pos = jnp.where(key == 0,
                    jnp.arange(E, dtype=jnp.int32) - c1,
                    n0 + c1 - 1)
    perm = jnp.zeros((E,), jnp.int32).at[pos].set(
        jnp.arange(E, dtype=jnp.int32), mode="drop", unique_indices=True)
    srcs = jnp.take(src, perm)
    dsts = jnp.take(dst, perm)
    cnts = jnp.broadcast_to(n0, (16,))
    pad = EPAD2 - E
    src_p = jnp.concatenate([srcs, jnp.zeros((pad,), jnp.int32)])
    dst_p = jnp.concatenate([dsts, jnp.full((pad,), TRASH, jnp.int32)])
    dst_2d = dst_p.reshape(EPAD2 // BATCH, BATCH)

    b0r = b0.reshape(1, -1)
    b1r = b1.reshape(1, -1)
    b2r = b2.reshape(1, -1)
    b3p = jnp.pad(b3, (0, 64)).reshape(1, -1)
    b4r = b4.reshape(1, 1)
    w3p = jnp.pad(W3, ((0, 0), (0, 64)))

    deg0, deg1 = _deg_kernel(dst_2d)

    x_p = jnp.pad(x, ((0, NROW - N), (0, 0)))
    g0 = _tc_call(_tca, [128], [], [128])(x_p, deg0, deg1)
    r0 = _agg_1(cnts, src_p, dst_2d, g0[0])

    g1 = _tc_call(_tcb, [128],
                  [(128, 1024), (1, 1024), (1024, 512)],
                  [128] * 4)(r0[0], deg0, deg1, W0, b0r, W1)
    r1 = _agg_4(cnts, src_p, dst_2d, *g1)

    g2 = _tc_call(_tcc, [128] * 4,
                  [(1, 512), (512, 256)],
                  [128, 128])(*r1, deg0, deg1, b1r, W2)
    r2 = _agg_2(cnts, src_p, dst_2d, *g2)

    g3 = _tc_call(_tcd, [128] * 2,
                  [(1, 256), (256, 128)],
                  [128])(*r2, deg0, deg1, b2r, w3p)
    r3 = _agg_1(cnts, src_p, dst_2d, g3[0])

    g4 = _tc_call(_tce, [128],
                  [(1, 128)],
                  [128])(r3[0], deg0, deg1, b3p)
    r4 = _agg_1(cnts, src_p, dst_2d, g4[0])

    out = _tc_call(_tcf, [128],
                   [(64, 1), (1, 1)],
                   [1])(r4[0], deg0, deg1, W4, b4r)
    return out[0][:N]


# speedup vs baseline: 1.0294x; 1.0294x over previous
import functools

import jax
import jax.numpy as jnp
from jax import lax
from jax.experimental import pallas as pl
from jax.experimental.pallas import tpu as pltpu
from jax.experimental.pallas import tpu_sc as plsc

N = 10000
E = 320000
NROW = 10240
TRASH = 10008
DC = 128
BATCH = 128
NB_ALL = 168
NB_HALF = 80
EPAD = 32 * NB_HALF * BATCH
NBTOT = EPAD // BATCH
EPAD2 = 2784 * BATCH
ROWS_PER_TILE = NROW // 16
DRAIN_CHUNK = 128

_mesh = plsc.VectorSubcoreMesh(core_axis_name="c", subcore_axis_name="s")


def _fill_rows(buf, nrows, ncols, value):
    vec = jnp.full((16,), value, jnp.float32)

    def body(i, _):
        for c in range(ncols // 16):
            buf[i, pl.ds(c * 16, 16)] = vec
        return 0

    lax.fori_loop(0, nrows, body, 0)


HALF = NROW // 2
ATRASH = 128
HROWS_PER_TILE = HALF // 16


def _chunk_pass(table, out, src_v, dst_v, bufs, acc, gsem,
                tile, cnt_b, row_base):
    for r0 in range(0, HROWS_PER_TILE, DRAIN_CHUNK):
        nrow = min(DRAIN_CHUNK, HROWS_PER_TILE - r0)
        row0 = tile * HROWS_PER_TILE + r0
        pltpu.sync_copy(table.at[pl.ds(row_base + row0, nrow)],
                        bufs[0].at[pl.ds(0, nrow)])
        pltpu.sync_copy(bufs[0].at[pl.ds(0, nrow)],
                        acc.at[pl.ds(row0, nrow)])
    plsc.subcore_barrier()

    rows_a, rows_b = bufs
    sem_a, sem_b = gsem

    def gather(j, buf, sem):
        return pltpu.make_async_copy(
            table.at[src_v.at[pl.ds(j * BATCH, BATCH)]], buf, sem)

    @pl.when(cnt_b > 0)
    def _():
        gather(0, rows_a, sem_a).start()

    def loop(k, _):
        ja = 2 * k
        jb = ja + 1
        gather(jb, rows_b, sem_b).start()
        gather(ja, rows_a, sem_a).wait()
        pltpu.sync_copy(rows_a, acc.at[dst_v.at[ja]], add=True)

        @pl.when(ja + 2 < cnt_b)
        def _():
            gather(ja + 2, rows_a, sem_a).start()

        gather(jb, rows_b, sem_b).wait()
        pltpu.sync_copy(rows_b, acc.at[dst_v.at[jb]], add=True)
        return 0

    lax.fori_loop(0, cnt_b // 2, loop, 0)

    @pl.when(cnt_b % 2 == 1)
    def _():
        jl = cnt_b - 1
        gather(jl, rows_a, sem_a).wait()
        pltpu.sync_copy(rows_a, acc.at[dst_v.at[jl]], add=True)

    plsc.subcore_barrier()

    for r0 in range(0, HROWS_PER_TILE, DRAIN_CHUNK):
        nrow = min(DRAIN_CHUNK, HROWS_PER_TILE - r0)
        row0 = tile * HROWS_PER_TILE + r0
        pltpu.sync_copy(acc.at[pl.ds(row0, nrow)],
                        bufs[0].at[pl.ds(0, nrow)])
        pltpu.sync_copy(bufs[0].at[pl.ds(0, nrow)],
                        out.at[pl.ds(row_base + row0, nrow)])


def _make_agg(num_chunks):
    @functools.partial(
        pl.kernel,
        out_type=[jax.ShapeDtypeStruct((NROW, DC), jnp.float32)]
        * num_chunks,
        mesh=_mesh,
        scratch_types=[
            pltpu.VMEM((16,), jnp.int32),
            pltpu.VMEM((NB_ALL * BATCH,), jnp.int32),
            pltpu.VMEM((NB_ALL, BATCH), jnp.int32),
            pltpu.VMEM((BATCH, DC), jnp.float32),
            pltpu.VMEM((BATCH, DC), jnp.float32),
            pltpu.VMEM_SHARED((HALF + ATRASH, DC), jnp.float32),
            pltpu.SemaphoreType.DMA,
            pltpu.SemaphoreType.DMA,
        ],
    )
    def agg(cnt_hbm, src_hbm, dst_hbm, *rest):
        tables = rest[:num_chunks]
        outs = rest[num_chunks:2 * num_chunks]
        (cnt_v, src_v, dst_v, rows_a, rows_b, acc, sem_a, sem_b) = \
            rest[2 * num_chunks:]
        core = lax.axis_index("c")
        tile = lax.axis_index("s")

        pltpu.sync_copy(cnt_hbm, cnt_v)
        n0 = cnt_v[...][0]
        lo8 = jnp.where(core == 0, 0, n0 // (8 * BATCH))
        hi = jnp.where(core == 0, (n0 + BATCH - 1) // BATCH, NBTOT)
        nbt8 = (hi - lo8 * 8 + 127) // 128
        lo_t = (lo8 + tile * nbt8) * 8
        cnt_t = jnp.maximum(0, jnp.minimum(nbt8 * 8, hi - lo_t))

        pltpu.sync_copy(src_hbm.at[pl.ds(lo_t * BATCH, NB_ALL * BATCH)],
                        src_v)
        pltpu.sync_copy(dst_hbm.at[pl.ds(lo_t, NB_ALL)], dst_v)

        row_base = core * HALF

        def remap(j, _):
            for c in range(BATCH // 16):
                d = dst_v[j, pl.ds(c * 16, 16)]
                local = d - row_base
                oob = (local < 0) | (local >= HALF)
                spread = HALF + tile * 8 + (local & 7)
                dst_v[j, pl.ds(c * 16, 16)] = jnp.where(oob, spread, local)
            return 0

        lax.fori_loop(0, NB_ALL, remap, 0)

        for cid in range(2):
            @pl.when(core == cid)
            def _(cid=cid):
                for ch in range(num_chunks):
                    _chunk_pass(tables[ch], outs[ch], src_v, dst_v,
                                (rows_a, rows_b), acc, (sem_a, sem_b),
                                tile, cnt_t, cid * HALF)

    return agg


def _make_deg():
    @functools.partial(
        pl.kernel,
        out_type=[jax.ShapeDtypeStruct((NROW, 16), jnp.float32)] * 2,
        mesh=_mesh,
        scratch_types=[
            pltpu.VMEM((NB_HALF, BATCH), jnp.int32),
            pltpu.VMEM((BATCH, 16), jnp.float32),
            pltpu.VMEM((BATCH, 16), jnp.float32),
            pltpu.VMEM_SHARED((NROW, 16), jnp.float32),
        ],
    )
    def deg(dst_hbm, out0, out1, dst_v, ones_v, stage_v, acc):
        core = lax.axis_index("c")
        tile = lax.axis_index("s")
        outs = (out0, out1)

        _fill_rows(ones_v, BATCH, 16, 1.0)

        base = (core * 16 + tile) * NB_HALF
        pltpu.sync_copy(dst_hbm.at[pl.ds(base, NB_HALF)], dst_v)

        vec = jnp.full((16,), 1.0, jnp.float32) * jnp.where(
            core == 0, 1.0, 0.0).astype(jnp.float32)

        def fill(i, _):
            stage_v[i, pl.ds(0, 16)] = vec
            return 0

        lax.fori_loop(0, BATCH, fill, 0)
        for r in range(ROWS_PER_TILE // DRAIN_CHUNK):
            row0 = tile * ROWS_PER_TILE + r * DRAIN_CHUNK
            pltpu.sync_copy(stage_v.at[pl.ds(0, DRAIN_CHUNK)],
                            acc.at[pl.ds(row0, DRAIN_CHUNK)])
        plsc.subcore_barrier()

        def loop(j, _):
            pltpu.sync_copy(ones_v, acc.at[dst_v.at[j]], add=True)
            return 0

        lax.fori_loop(0, NB_HALF, loop, 0)
        plsc.subcore_barrier()

        for cid in range(2):
            @pl.when(core == cid)
            def _(cid=cid):
                for r in range(ROWS_PER_TILE // DRAIN_CHUNK):
                    row0 = tile * ROWS_PER_TILE + r * DRAIN_CHUNK
                    pltpu.sync_copy(acc.at[pl.ds(row0, DRAIN_CHUNK)],
                                    stage_v.at[pl.ds(0, DRAIN_CHUNK)])
                    pltpu.sync_copy(stage_v.at[pl.ds(0, DRAIN_CHUNK)],
                                    outs[cid].at[pl.ds(row0, DRAIN_CHUNK)])

    return deg


_agg_1 = _make_agg(1)
_agg_4 = _make_agg(4)
_agg_2 = _make_agg(2)
_deg_kernel = _make_deg()


BM = 1024


def _row(d):
    return pl.BlockSpec((BM, d), lambda i: (i, 0))


def _full(shape):
    return pl.BlockSpec(shape, lambda i: tuple(0 for _ in shape))


def _tc_call(body, in_dims, full_shapes, out_dims):
    in_specs = ([_row(d) for d in in_dims]
                + [_row(16), _row(16)]
                + [_full(s) for s in full_shapes])
    return pl.pallas_call(
        body,
        grid=(NROW // BM,),
        in_specs=in_specs,
        out_specs=[_row(d) for d in out_dims],
        out_shape=[jax.ShapeDtypeStruct((NROW, d), jnp.float32)
                   for d in out_dims],
    )


def _scale(d0_ref, d1_ref):
    return lax.rsqrt(d0_ref[...] + d1_ref[...])[:, 0:1]


def _tca(x_ref, d0, d1, o0):
    o0[...] = x_ref[...] * _scale(d0, d1)


def _tcb(r0, d0, d1, w0, b0, w1, o0, o1, o2, o3):
    s = _scale(d0, d1)
    h = jnp.maximum(
        jnp.dot(r0[...] * s, w0[...], preferred_element_type=jnp.float32)
        + b0[...], 0.0)
    z = jnp.dot(h, w1[...], preferred_element_type=jnp.float32) * s
    for j, o in enumerate((o0, o1, o2, o3)):
        o[...] = z[:, j * 128:(j + 1) * 128]


def _tcc(r0, r1, r2, r3, d0, d1, b1, w2, o0, o1):
    s = _scale(d0, d1)
    z = None
    for j, r in enumerate((r0, r1, r2, r3)):
        h = jnp.maximum(r[...] * s + b1[..., j * 128:(j + 1) * 128], 0.0)
        p = jnp.dot(h, w2[j * 128:(j + 1) * 128, :],
                    preferred_element_type=jnp.float32)
        z = p if z is None else z + p
    z = z * s
    o0[...] = z[:, :128]
    o1[...] = z[:, 128:]


def _tcd(r0, r1, d0, d1, b2, w3p, o0):
    s = _scale(d0, d1)
    z = None
    for j, r in enumerate((r0, r1)):
        h = jnp.maximum(r[...] * s + b2[..., j * 128:(j + 1) * 128], 0.0)
        p = jnp.dot(h, w3p[j * 128:(j + 1) * 128, :],
                    preferred_element_type=jnp.float32)
        z = p if z is None else z + p
    o0[...] = z * s


def _tce(ra, d0, d1, b3p, o0):
    s = _scale(d0, d1)
    o0[...] = jnp.maximum(ra[...] * s + b3p[...], 0.0) * s


def _tcf(ra, d0, d1, w4, b4, o0):
    s = _scale(d0, d1)
    o0[...] = jnp.dot((ra[...] * s)[:, :64], w4[...],
                      preferred_element_type=jnp.float32) + b4[...]


def kernel(x, edge_index, W0, b0, W1, b1, W2, b2, W3, b3, W4, b4):
    src = edge_index[0]
    dst = edge_index[1]
    key = (dst >= HALF).astype(jnp.int32)
    n0 = (jnp.int32(E) - jnp.sum(key)).astype(jnp.int32)
    perm = jnp.argsort(key, stable=True).astype(jnp.int32)
    srcs = jnp.take(src, perm)
    dsts = jnp.take(dst, perm)
    cnts = jnp.broadcast_to(n0, (16,))
    pad = EPAD2 - E
    src_p = jnp.concatenate([srcs, jnp.zeros((pad,), jnp.int32)])
    dst_p = jnp.concatenate([dsts, jnp.full((pad,), TRASH, jnp.int32)])
    dst_2d = dst_p.reshape(EPAD2 // BATCH, BATCH)

    b0r = b0.reshape(1, -1)
    b1r = b1.reshape(1, -1)
    b2r = b2.reshape(1, -1)
    b3p = jnp.pad(b3, (0, 64)).reshape(1, -1)
    b4r = b4.reshape(1, 1)
    w3p = jnp.pad(W3, ((0, 0), (0, 64)))

    deg0, deg1 = _deg_kernel(dst_2d)

    x_p = jnp.pad(x, ((0, NROW - N), (0, 0)))
    g0 = _tc_call(_tca, [128], [], [128])(x_p, deg0, deg1)
    r0 = _agg_1(cnts, src_p, dst_2d, g0[0])

    g1 = _tc_call(_tcb, [128],
                  [(128, 1024), (1, 1024), (1024, 512)],
                  [128] * 4)(r0[0], deg0, deg1, W0, b0r, W1)
    r1 = _agg_4(cnts, src_p, dst_2d, *g1)

    g2 = _tc_call(_tcc, [128] * 4,
                  [(1, 512), (512, 256)],
                  [128, 128])(*r1, deg0, deg1, b1r, W2)
    r2 = _agg_2(cnts, src_p, dst_2d, *g2)

    g3 = _tc_call(_tcd, [128] * 2,
                  [(1, 256), (256, 128)],
                  [128])(*r2, deg0, deg1, b2r, w3p)
    r3 = _agg_1(cnts, src_p, dst_2d, g3[0])

    g4 = _tc_call(_tce, [128],
                  [(1, 128)],
                  [128])(r3[0], deg0, deg1, b3p)
    r4 = _agg_1(cnts, src_p, dst_2d, g4[0])

    out = _tc_call(_tcf, [128],
                   [(64, 1), (1, 1)],
                   [1])(r4[0], deg0, deg1, W4, b4r)
    return out[0][:N]


# speedup vs baseline: 1.0372x; 1.0076x over previous
import functools

import jax
import jax.numpy as jnp
from jax import lax
from jax.experimental import pallas as pl
from jax.experimental.pallas import tpu as pltpu
from jax.experimental.pallas import tpu_sc as plsc

N = 10000
E = 320000
NROW = 10240
TRASH = 10008
DC = 128
BATCH = 128
NB_ALL = 168
NB_HALF = 80
EPAD = 32 * NB_HALF * BATCH
NBTOT = EPAD // BATCH
EPAD2 = 2784 * BATCH
ROWS_PER_TILE = NROW // 16
DRAIN_CHUNK = 128

_mesh = plsc.VectorSubcoreMesh(core_axis_name="c", subcore_axis_name="s")


def _fill_rows(buf, nrows, ncols, value):
    vec = jnp.full((16,), value, jnp.float32)

    def body(i, _):
        for c in range(ncols // 16):
            buf[i, pl.ds(c * 16, 16)] = vec
        return 0

    lax.fori_loop(0, nrows, body, 0)


HALF = NROW // 2
ATRASH = 128
HROWS_PER_TILE = HALF // 16


def _chunk_pass(table, out, src_v, dst_v, bufs, acc, gsem,
                tile, cnt_b, row_base):
    for r0 in range(0, HROWS_PER_TILE, DRAIN_CHUNK):
        nrow = min(DRAIN_CHUNK, HROWS_PER_TILE - r0)
        row0 = tile * HROWS_PER_TILE + r0
        pltpu.sync_copy(table.at[pl.ds(row_base + row0, nrow)],
                        bufs[0].at[pl.ds(0, nrow)])
        pltpu.sync_copy(bufs[0].at[pl.ds(0, nrow)],
                        acc.at[pl.ds(row0, nrow)])
    plsc.subcore_barrier()

    rows_a, rows_b = bufs
    sem_a, sem_b = gsem

    def gather(j, buf, sem):
        return pltpu.make_async_copy(
            table.at[src_v.at[pl.ds(j * BATCH, BATCH)]], buf, sem)

    @pl.when(cnt_b > 0)
    def _():
        gather(0, rows_a, sem_a).start()

    def loop(k, _):
        ja = 2 * k
        jb = ja + 1
        gather(jb, rows_b, sem_b).start()
        gather(ja, rows_a, sem_a).wait()
        pltpu.sync_copy(rows_a, acc.at[dst_v.at[ja]], add=True)

        @pl.when(ja + 2 < cnt_b)
        def _():
            gather(ja + 2, rows_a, sem_a).start()

        gather(jb, rows_b, sem_b).wait()
        pltpu.sync_copy(rows_b, acc.at[dst_v.at[jb]], add=True)
        return 0

    lax.fori_loop(0, cnt_b // 2, loop, 0)

    @pl.when(cnt_b % 2 == 1)
    def _():
        jl = cnt_b - 1
        gather(jl, rows_a, sem_a).wait()
        pltpu.sync_copy(rows_a, acc.at[dst_v.at[jl]], add=True)

    plsc.subcore_barrier()

    for r0 in range(0, HROWS_PER_TILE, DRAIN_CHUNK):
        nrow = min(DRAIN_CHUNK, HROWS_PER_TILE - r0)
        row0 = tile * HROWS_PER_TILE + r0
        pltpu.sync_copy(acc.at[pl.ds(row0, nrow)],
                        bufs[0].at[pl.ds(0, nrow)])
        pltpu.sync_copy(bufs[0].at[pl.ds(0, nrow)],
                        out.at[pl.ds(row_base + row0, nrow)])


def _make_agg(num_chunks):
    @functools.partial(
        pl.kernel,
        out_type=[jax.ShapeDtypeStruct((NROW, DC), jnp.float32)]
        * num_chunks,
        mesh=_mesh,
        scratch_types=[
            pltpu.VMEM((16,), jnp.int32),
            pltpu.VMEM((NB_ALL * BATCH,), jnp.int32),
            pltpu.VMEM((NB_ALL, BATCH), jnp.int32),
            pltpu.VMEM((BATCH, DC), jnp.float32),
            pltpu.VMEM((BATCH, DC), jnp.float32),
            pltpu.VMEM_SHARED((HALF + ATRASH, DC), jnp.float32),
            pltpu.SemaphoreType.DMA,
            pltpu.SemaphoreType.DMA,
        ],
    )
    def agg(cnt_hbm, src_hbm, dst_hbm, *rest):
        tables = rest[:num_chunks]
        outs = rest[num_chunks:2 * num_chunks]
        (cnt_v, src_v, dst_v, rows_a, rows_b, acc, sem_a, sem_b) = \
            rest[2 * num_chunks:]
        core = lax.axis_index("c")
        tile = lax.axis_index("s")

        pltpu.sync_copy(cnt_hbm, cnt_v)
        n0 = cnt_v[...][0]
        lo8 = jnp.where(core == 0, 0, n0 // (8 * BATCH))
        hi = jnp.where(core == 0, (n0 + BATCH - 1) // BATCH, NBTOT)
        nbt8 = (hi - lo8 * 8 + 127) // 128
        lo_t = (lo8 + tile * nbt8) * 8
        cnt_t = jnp.maximum(0, jnp.minimum(nbt8 * 8, hi - lo_t))

        pltpu.sync_copy(src_hbm.at[pl.ds(lo_t * BATCH, NB_ALL * BATCH)],
                        src_v)
        pltpu.sync_copy(dst_hbm.at[pl.ds(lo_t, NB_ALL)], dst_v)

        row_base = core * HALF

        def remap(j, _):
            for c in range(BATCH // 16):
                d = dst_v[j, pl.ds(c * 16, 16)]
                local = d - row_base
                oob = (local < 0) | (local >= HALF)
                spread = HALF + tile * 8 + (local & 7)
                dst_v[j, pl.ds(c * 16, 16)] = jnp.where(oob, spread, local)
            return 0

        lax.fori_loop(0, NB_ALL, remap, 0)

        for cid in range(2):
            @pl.when(core == cid)
            def _(cid=cid):
                for ch in range(num_chunks):
                    _chunk_pass(tables[ch], outs[ch], src_v, dst_v,
                                (rows_a, rows_b), acc, (sem_a, sem_b),
                                tile, cnt_t, cid * HALF)

    return agg


def _make_deg():
    @functools.partial(
        pl.kernel,
        out_type=[jax.ShapeDtypeStruct((NROW, 16), jnp.float32)] * 2,
        mesh=_mesh,
        scratch_types=[
            pltpu.VMEM((NB_HALF, BATCH), jnp.int32),
            pltpu.VMEM((BATCH, 16), jnp.float32),
            pltpu.VMEM((BATCH, 16), jnp.float32),
            pltpu.VMEM_SHARED((NROW, 16), jnp.float32),
        ],
    )
    def deg(dst_hbm, out0, out1, dst_v, ones_v, stage_v, acc):
        core = lax.axis_index("c")
        tile = lax.axis_index("s")
        outs = (out0, out1)

        _fill_rows(ones_v, BATCH, 16, 1.0)

        base = (core * 16 + tile) * NB_HALF
        pltpu.sync_copy(dst_hbm.at[pl.ds(base, NB_HALF)], dst_v)

        vec = jnp.full((16,), 1.0, jnp.float32) * jnp.where(
            core == 0, 1.0, 0.0).astype(jnp.float32)

        def fill(i, _):
            stage_v[i, pl.ds(0, 16)] = vec
            return 0

        lax.fori_loop(0, BATCH, fill, 0)
        for r in range(ROWS_PER_TILE // DRAIN_CHUNK):
            row0 = tile * ROWS_PER_TILE + r * DRAIN_CHUNK
            pltpu.sync_copy(stage_v.at[pl.ds(0, DRAIN_CHUNK)],
                            acc.at[pl.ds(row0, DRAIN_CHUNK)])
        plsc.subcore_barrier()

        def loop(j, _):
            pltpu.sync_copy(ones_v, acc.at[dst_v.at[j]], add=True)
            return 0

        lax.fori_loop(0, NB_HALF, loop, 0)
        plsc.subcore_barrier()

        for cid in range(2):
            @pl.when(core == cid)
            def _(cid=cid):
                for r in range(ROWS_PER_TILE // DRAIN_CHUNK):
                    row0 = tile * ROWS_PER_TILE + r * DRAIN_CHUNK
                    pltpu.sync_copy(acc.at[pl.ds(row0, DRAIN_CHUNK)],
                                    stage_v.at[pl.ds(0, DRAIN_CHUNK)])
                    pltpu.sync_copy(stage_v.at[pl.ds(0, DRAIN_CHUNK)],
                                    outs[cid].at[pl.ds(row0, DRAIN_CHUNK)])

    return deg


_agg_1 = _make_agg(1)
_agg_4 = _make_agg(4)
_agg_2 = _make_agg(2)
_deg_kernel = _make_deg()


BM = 1024


def _row(d):
    return pl.BlockSpec((BM, d), lambda i: (i, 0))


def _full(shape):
    return pl.BlockSpec(shape, lambda i: tuple(0 for _ in shape))


def _tc_call(body, in_dims, full_shapes, out_dims):
    in_specs = ([_row(d) for d in in_dims]
                + [_row(16), _row(16)]
                + [_full(s) for s in full_shapes])
    return pl.pallas_call(
        body,
        grid=(NROW // BM,),
        in_specs=in_specs,
        out_specs=[_row(d) for d in out_dims],
        out_shape=[jax.ShapeDtypeStruct((NROW, d), jnp.float32)
                   for d in out_dims],
    )


def _scale(d0_ref, d1_ref):
    return lax.rsqrt(d0_ref[...] + d1_ref[...])[:, 0:1]


def _tca(x_ref, d0, d1, o0):
    o0[...] = x_ref[...] * _scale(d0, d1)


def _tcb(r0, d0, d1, w0, b0, w1, o0, o1, o2, o3):
    s = _scale(d0, d1)
    h = jnp.maximum(
        jnp.dot(r0[...] * s, w0[...], preferred_element_type=jnp.float32)
        + b0[...], 0.0)
    z = jnp.dot(h, w1[...], preferred_element_type=jnp.float32) * s
    for j, o in enumerate((o0, o1, o2, o3)):
        o[...] = z[:, j * 128:(j + 1) * 128]


def _tcc(r0, r1, r2, r3, d0, d1, b1, w2, o0, o1):
    s = _scale(d0, d1)
    z = None
    for j, r in enumerate((r0, r1, r2, r3)):
        h = jnp.maximum(r[...] * s + b1[..., j * 128:(j + 1) * 128], 0.0)
        p = jnp.dot(h, w2[j * 128:(j + 1) * 128, :],
                    preferred_element_type=jnp.float32)
        z = p if z is None else z + p
    z = z * s
    o0[...] = z[:, :128]
    o1[...] = z[:, 128:]


def _tcd(r0, r1, d0, d1, b2, w3p, o0):
    s = _scale(d0, d1)
    z = None
    for j, r in enumerate((r0, r1)):
        h = jnp.maximum(r[...] * s + b2[..., j * 128:(j + 1) * 128], 0.0)
        p = jnp.dot(h, w3p[j * 128:(j + 1) * 128, :],
                    preferred_element_type=jnp.float32)
        z = p if z is None else z + p
    o0[...] = z * s


def _tce(ra, d0, d1, b3p, o0):
    s = _scale(d0, d1)
    o0[...] = jnp.maximum(ra[...] * s + b3p[...], 0.0) * s


def _tcf(ra, d0, d1, w4, b4, o0):
    s = _scale(d0, d1)
    o0[...] = jnp.dot((ra[...] * s)[:, :64], w4[...],
                      preferred_element_type=jnp.float32) + b4[...]


def kernel(x, edge_index, W0, b0, W1, b1, W2, b2, W3, b3, W4, b4):
    src = edge_index[0]
    dst = edge_index[1]
    key = (dst >= HALF).astype(jnp.int32)
    n0 = (jnp.int32(E) - jnp.sum(key)).astype(jnp.int32)
    packed = (key << 19) | jnp.arange(E, dtype=jnp.int32)
    perm = jnp.sort(packed) & ((1 << 19) - 1)
    srcs = jnp.take(src, perm)
    dsts = jnp.take(dst, perm)
    cnts = jnp.broadcast_to(n0, (16,))
    pad = EPAD2 - E
    src_p = jnp.concatenate([srcs, jnp.zeros((pad,), jnp.int32)])
    dst_p = jnp.concatenate([dsts, jnp.full((pad,), TRASH, jnp.int32)])
    dst_2d = dst_p.reshape(EPAD2 // BATCH, BATCH)

    b0r = b0.reshape(1, -1)
    b1r = b1.reshape(1, -1)
    b2r = b2.reshape(1, -1)
    b3p = jnp.pad(b3, (0, 64)).reshape(1, -1)
    b4r = b4.reshape(1, 1)
    w3p = jnp.pad(W3, ((0, 0), (0, 64)))

    deg0, deg1 = _deg_kernel(dst_2d)

    x_p = jnp.pad(x, ((0, NROW - N), (0, 0)))
    g0 = _tc_call(_tca, [128], [], [128])(x_p, deg0, deg1)
    r0 = _agg_1(cnts, src_p, dst_2d, g0[0])

    g1 = _tc_call(_tcb, [128],
                  [(128, 1024), (1, 1024), (1024, 512)],
                  [128] * 4)(r0[0], deg0, deg1, W0, b0r, W1)
    r1 = _agg_4(cnts, src_p, dst_2d, *g1)

    g2 = _tc_call(_tcc, [128] * 4,
                  [(1, 512), (512, 256)],
                  [128, 128])(*r1, deg0, deg1, b1r, W2)
    r2 = _agg_2(cnts, src_p, dst_2d, *g2)

    g3 = _tc_call(_tcd, [128] * 2,
                  [(1, 256), (256, 128)],
                  [128])(*r2, deg0, deg1, b2r, w3p)
    r3 = _agg_1(cnts, src_p, dst_2d, g3[0])

    g4 = _tc_call(_tce, [128],
                  [(1, 128)],
                  [128])(r3[0], deg0, deg1, b3p)
    r4 = _agg_1(cnts, src_p, dst_2d, g4[0])

    out = _tc_call(_tcf, [128],
                   [(64, 1), (1, 1)],
                   [1])(r4[0], deg0, deg1, W4, b4r)
    return out[0][:N]
